# baseline (speedup 1.0000x reference)
import functools

import jax
import jax.numpy as jnp
from jax import lax
from jax.experimental import pallas as pl
from jax.experimental.pallas import tpu as pltpu
from jax.experimental.pallas import tpu_sc as plsc

N_POINTS = 1048576
N_FEATURES = 4
DIM = 3
HASH_MASK = 4194304 - 1
RESOLUTION = 512.0
P1 = jnp.int32(2654435761 - (1 << 32))
P2 = jnp.int32(805459861)

NC, NS, L = 2, 16, 16
NW = NC * NS
PW = N_POINTS // NW
T = 1024
N_CHUNKS = PW // T
GB = T // 128
N_CHUNK_ROWS = 4194304 * N_FEATURES // 8


def _body(x_hbm, table_hbm, out_hbm,
          xvA, xvB, hvA, hvB, idxA, idxB, rvA, rvB, outvA, outvB,
          semA, semB, semSA, semSB, semOA, semOB):
    i32 = jnp.int32
    wid = (lax.axis_index("s") * i32(NC) + lax.axis_index("c")).astype(i32)
    mask = jnp.full((L,), HASH_MASK, i32)
    m15 = jnp.full((L,), 15, i32)
    m7 = jnp.full((L,), 7, i32)
    res = jnp.float32(RESOLUTION)
    lanes = lax.iota(i32, L)

    def stage(xv, base, semS):
        pltpu.async_copy(x_hbm.at[:, pl.ds(base, T)], xv, semS)

    def stage_wait(xv, semS):
        pltpu.make_async_copy(x_hbm.at[:, pl.ds(i32(0), T)], xv, semS).wait()

    def hashc(xv, hv, idxv):
        def hash_step(i, _):
            for k in range(128 // L):
                off = i * i32(128) + i32(k * L)
                x0 = xv[i32(0), pl.ds(off, L)]
                x1 = xv[i32(1), pl.ds(off, L)]
                x2 = xv[i32(2), pl.ds(off, L)]
                i0 = (x0 * res).astype(i32)
                i1 = (x1 * res).astype(i32)
                i2 = (x2 * res).astype(i32)
                h = (i0 ^ (i1 * P1) ^ (i2 * P2)) & mask
                hv[i, pl.ds(k * L, L)] = h
                q0 = (lax.shift_right_logical(h, i32(7)) * i32(64)
                      + (lax.shift_right_logical(h, i32(3)) & m15))
                for c in range(N_FEATURES):
                    idxv[i32(c * GB) + i, pl.ds(k * L, L)] = q0 + i32(c * 16)
            return _

        lax.fori_loop(i32(0), i32(GB), hash_step, i32(0))

    def fire(idxv, rv, sem):
        def fire_step(j, _):
            for c in range(N_FEATURES):
                pltpu.async_copy(
                    table_hbm.at[idxv.at[i32(c * GB) + j]],
                    rv.at[pl.ds(i32(c * T) + j * i32(128), 128), :],
                    sem,
                )
            return _

        lax.fori_loop(i32(0), i32(GB), fire_step, i32(0))

    def drain(rv, sem):
        pltpu.make_async_copy(
            table_hbm.at[pl.ds(i32(0), N_FEATURES * T), :], rv, sem
        ).wait()

    def extract(hv, rv, outv):
        def ex_step(i, _):
            jloc = i * i32(L) + lanes
            h16 = hv[lax.shift_right_logical(i, i32(3)),
                     pl.ds(lax.rem(i, i32(8)) * i32(L), L)]
            sub = h16 & m7
            obase = (lax.div(i, i32(8)) * i32(512)
                     + lax.rem(i, i32(8)) * i32(L))
            for c in range(N_FEATURES):
                val = plsc.load_gather(rv, [i32(c * T) + jloc, sub])
                outv[pl.ds(obase + i32(c * 128), L)] = val
            return _

        lax.fori_loop(i32(0), i32(T // L), ex_step, i32(0))

    def outdma(outv, base, sem):
        pltpu.async_copy(outv, out_hbm.at[pl.ds(base * i32(N_FEATURES),
                                                T * N_FEATURES)], sem)

    def outdma_wait(sem):
        pltpu.make_async_copy(
            outvA, out_hbm.at[pl.ds(i32(0), T * N_FEATURES)], sem
        ).wait()

    def pair_step(tt, _):
        base_e = wid * i32(PW) + tt * i32(2 * T)
        base_o = base_e + i32(T)

        stage(xvA, base_e, semSA)
        stage(xvB, base_o, semSB)
        stage_wait(xvA, semSA)
        hashc(xvA, hvA, idxA)

        @pl.when(tt > i32(1))
        def _wait_prev_outB():
            outdma_wait(semOB)

        @pl.when(tt > i32(0))
        def _finish_prev_odd():
            drain(rvB, semB)
            extract(hvB, rvB, outvB)
            outdma(outvB, base_e - i32(T), semOB)

        fire(idxA, rvA, semA)

        stage_wait(xvB, semSB)
        hashc(xvB, hvB, idxB)

        drain(rvA, semA)

        @pl.when(tt > i32(0))
        def _wait_prev_outA():
            outdma_wait(semOA)

        extract(hvA, rvA, outvA)
        outdma(outvA, base_e, semOA)

        fire(idxB, rvB, semB)
        return _

    lax.fori_loop(jnp.int32(0), jnp.int32(N_CHUNKS // 2), pair_step,
                  jnp.int32(0))
    last_base = wid * i32(PW) + i32((N_CHUNKS - 1) * T)
    drain(rvB, semB)

    @pl.when(jnp.int32(N_CHUNKS // 2) > i32(1))
    def _wait_last_outB():
        outdma_wait(semOB)

    outdma_wait(semOA)
    extract(hvB, rvB, outvB)
    outdma(outvB, last_base, semOB)
    outdma_wait(semOB)


@jax.jit
def _run(x, table):
    kfn = functools.partial(
        pl.kernel,
        mesh=plsc.VectorSubcoreMesh(core_axis_name="c", subcore_axis_name="s"),
        compiler_params=pltpu.CompilerParams(
            use_tc_tiling_on_sc=False, needs_layout_passes=False),
        out_type=jax.ShapeDtypeStruct((N_POINTS * N_FEATURES,), jnp.float32),
        scratch_types=[
            pltpu.VMEM((DIM, T), jnp.float32),
            pltpu.VMEM((DIM, T), jnp.float32),
            pltpu.VMEM((GB, 128), jnp.int32),
            pltpu.VMEM((GB, 128), jnp.int32),
            pltpu.VMEM((N_FEATURES * GB, 128), jnp.int32),
            pltpu.VMEM((N_FEATURES * GB, 128), jnp.int32),
            pltpu.VMEM((N_FEATURES * T, 8), jnp.float32),
            pltpu.VMEM((N_FEATURES * T, 8), jnp.float32),
            pltpu.VMEM((T * N_FEATURES,), jnp.float32),
            pltpu.VMEM((T * N_FEATURES,), jnp.float32),
            pltpu.SemaphoreType.DMA,
            pltpu.SemaphoreType.DMA,
            pltpu.SemaphoreType.DMA,
            pltpu.SemaphoreType.DMA,
            pltpu.SemaphoreType.DMA,
            pltpu.SemaphoreType.DMA,
        ],
    )(_body)
    xt = x.T
    chunks = jnp.transpose(
        table.reshape(32768, 128, N_FEATURES), (0, 2, 1)
    ).reshape(N_CHUNK_ROWS, 8)
    out1d = kfn(xt, chunks)
    return jnp.transpose(
        out1d.reshape(N_POINTS // 128, N_FEATURES, 128), (0, 2, 1)
    ).reshape(N_POINTS, N_FEATURES)


def kernel(x, table):
    return _run(x, table)


# speedup vs baseline: 1.1683x; 1.1683x over previous
import functools

import jax
import jax.numpy as jnp
from jax import lax
from jax.experimental import pallas as pl
from jax.experimental.pallas import tpu as pltpu
from jax.experimental.pallas import tpu_sc as plsc

N_POINTS = 1048576
N_FEATURES = 4
DIM = 3
HASH_MASK = 4194304 - 1
RESOLUTION = 512.0
P1 = jnp.int32(2654435761 - (1 << 32))
P2 = jnp.int32(805459861)

NC, NS, L = 2, 16, 16
NW = NC * NS
PW = N_POINTS // NW
T = 1024
N_CHUNKS = PW // T
GB = T // 128
N_CHUNK_ROWS = 4194304 * N_FEATURES // 8


def _body(x_hbm, table_hbm, out_hbm,
          xvA, xvB, hvA, hvB, idxA, idxB, rvA, rvB, outvA, outvB,
          semA, semB, semSA, semSB, semOA, semOB):
    i32 = jnp.int32
    wid = (lax.axis_index("s") * i32(NC) + lax.axis_index("c")).astype(i32)
    mask = jnp.full((L,), HASH_MASK, i32)
    m15 = jnp.full((L,), 15, i32)
    m7 = jnp.full((L,), 7, i32)
    res = jnp.float32(RESOLUTION)
    lanes = lax.iota(i32, L)

    def stage(xv, base, semS):
        pltpu.async_copy(x_hbm.at[:, pl.ds(base, T)], xv, semS)

    def stage_wait(xv, semS):
        pltpu.make_async_copy(x_hbm.at[:, pl.ds(i32(0), T)], xv, semS).wait()

    def hashc(xv, hv, idxv):
        @plsc.parallel_loop(jnp.int32(0), jnp.int32(GB), jnp.int32(1), unroll=2)
        def hash_step(i):
            for k in range(128 // L):
                off = i * i32(128) + i32(k * L)
                x0 = xv[i32(0), pl.ds(off, L)]
                x1 = xv[i32(1), pl.ds(off, L)]
                x2 = xv[i32(2), pl.ds(off, L)]
                i0 = (x0 * res).astype(i32)
                i1 = (x1 * res).astype(i32)
                i2 = (x2 * res).astype(i32)
                h = (i0 ^ (i1 * P1) ^ (i2 * P2)) & mask
                hv[i, pl.ds(k * L, L)] = h
                q0 = (lax.shift_right_logical(h, i32(7)) * i32(64)
                      + (lax.shift_right_logical(h, i32(3)) & m15))
                for c in range(N_FEATURES):
                    idxv[i32(c * GB) + i, pl.ds(k * L, L)] = q0 + i32(c * 16)

    def fire(idxv, rv, sem):
        def fire_step(j, _):
            for c in range(N_FEATURES):
                pltpu.async_copy(
                    table_hbm.at[idxv.at[i32(c * GB) + j]],
                    rv.at[pl.ds(i32(c * T) + j * i32(128), 128), :],
                    sem,
                )
            return _

        lax.fori_loop(i32(0), i32(GB), fire_step, i32(0))

    def drain(rv, sem):
        pltpu.make_async_copy(
            table_hbm.at[pl.ds(i32(0), N_FEATURES * T), :], rv, sem
        ).wait()

    def extract(hv, rv, outv):
        @plsc.parallel_loop(jnp.int32(0), jnp.int32(T // L), jnp.int32(1), unroll=2)
        def ex_step(i):
            jloc = i * i32(L) + lanes
            h16 = hv[lax.shift_right_logical(i, i32(3)),
                     pl.ds(lax.rem(i, i32(8)) * i32(L), L)]
            sub = h16 & m7
            obase = (lax.div(i, i32(8)) * i32(512)
                     + lax.rem(i, i32(8)) * i32(L))
            for c in range(N_FEATURES):
                val = plsc.load_gather(rv, [i32(c * T) + jloc, sub])
                outv[pl.ds(obase + i32(c * 128), L)] = val

    def outdma(outv, base, sem):
        pltpu.async_copy(outv, out_hbm.at[pl.ds(base * i32(N_FEATURES),
                                                T * N_FEATURES)], sem)

    def outdma_wait(sem):
        pltpu.make_async_copy(
            outvA, out_hbm.at[pl.ds(i32(0), T * N_FEATURES)], sem
        ).wait()

    def pair_step(tt, _):
        base_e = wid * i32(PW) + tt * i32(2 * T)
        base_o = base_e + i32(T)

        stage(xvA, base_e, semSA)
        stage(xvB, base_o, semSB)
        stage_wait(xvA, semSA)
        hashc(xvA, hvA, idxA)

        @pl.when(tt > i32(1))
        def _wait_prev_outB():
            outdma_wait(semOB)

        @pl.when(tt > i32(0))
        def _finish_prev_odd():
            drain(rvB, semB)
            extract(hvB, rvB, outvB)
            outdma(outvB, base_e - i32(T), semOB)

        fire(idxA, rvA, semA)

        stage_wait(xvB, semSB)
        hashc(xvB, hvB, idxB)

        drain(rvA, semA)

        @pl.when(tt > i32(0))
        def _wait_prev_outA():
            outdma_wait(semOA)

        extract(hvA, rvA, outvA)
        outdma(outvA, base_e, semOA)

        fire(idxB, rvB, semB)
        return _

    lax.fori_loop(jnp.int32(0), jnp.int32(N_CHUNKS // 2), pair_step,
                  jnp.int32(0))
    last_base = wid * i32(PW) + i32((N_CHUNKS - 1) * T)
    drain(rvB, semB)

    @pl.when(jnp.int32(N_CHUNKS // 2) > i32(1))
    def _wait_last_outB():
        outdma_wait(semOB)

    outdma_wait(semOA)
    extract(hvB, rvB, outvB)
    outdma(outvB, last_base, semOB)
    outdma_wait(semOB)


@jax.jit
def _run(x, table):
    kfn = functools.partial(
        pl.kernel,
        mesh=plsc.VectorSubcoreMesh(core_axis_name="c", subcore_axis_name="s"),
        compiler_params=pltpu.CompilerParams(
            use_tc_tiling_on_sc=False, needs_layout_passes=False),
        out_type=jax.ShapeDtypeStruct((N_POINTS * N_FEATURES,), jnp.float32),
        scratch_types=[
            pltpu.VMEM((DIM, T), jnp.float32),
            pltpu.VMEM((DIM, T), jnp.float32),
            pltpu.VMEM((GB, 128), jnp.int32),
            pltpu.VMEM((GB, 128), jnp.int32),
            pltpu.VMEM((N_FEATURES * GB, 128), jnp.int32),
            pltpu.VMEM((N_FEATURES * GB, 128), jnp.int32),
            pltpu.VMEM((N_FEATURES * T, 8), jnp.float32),
            pltpu.VMEM((N_FEATURES * T, 8), jnp.float32),
            pltpu.VMEM((T * N_FEATURES,), jnp.float32),
            pltpu.VMEM((T * N_FEATURES,), jnp.float32),
            pltpu.SemaphoreType.DMA,
            pltpu.SemaphoreType.DMA,
            pltpu.SemaphoreType.DMA,
            pltpu.SemaphoreType.DMA,
            pltpu.SemaphoreType.DMA,
            pltpu.SemaphoreType.DMA,
        ],
    )(_body)
    xt = x.T
    chunks = jnp.transpose(
        table.reshape(32768, 128, N_FEATURES), (0, 2, 1)
    ).reshape(N_CHUNK_ROWS, 8)
    out1d = kfn(xt, chunks)
    return jnp.transpose(
        out1d.reshape(N_POINTS // 128, N_FEATURES, 128), (0, 2, 1)
    ).reshape(N_POINTS, N_FEATURES)


def kernel(x, table):
    return _run(x, table)


# speedup vs baseline: 1.1718x; 1.0030x over previous
import functools

import jax
import jax.numpy as jnp
from jax import lax
from jax.experimental import pallas as pl
from jax.experimental.pallas import tpu as pltpu
from jax.experimental.pallas import tpu_sc as plsc

N_POINTS = 1048576
N_FEATURES = 4
DIM = 3
HASH_MASK = 4194304 - 1
RESOLUTION = 512.0
P1 = jnp.int32(2654435761 - (1 << 32))
P2 = jnp.int32(805459861)

NC, NS, L = 2, 16, 16
NW = NC * NS
PW = N_POINTS // NW
T = 1024
N_CHUNKS = PW // T
GB = T // 128
N_CHUNK_ROWS = 4194304 * N_FEATURES // 8


def _body(x_hbm, table_hbm, out_hbm,
          xvA, xvB, hvA, hvB, idxA, idxB, rvA, rvB, outvA, outvB,
          semA, semB, semSA, semSB, semOA, semOB):
    i32 = jnp.int32
    wid = (lax.axis_index("s") * i32(NC) + lax.axis_index("c")).astype(i32)
    mask = jnp.full((L,), HASH_MASK, i32)
    m15 = jnp.full((L,), 15, i32)
    m7 = jnp.full((L,), 7, i32)
    res = jnp.float32(RESOLUTION)
    lanes = lax.iota(i32, L)

    def stage(xv, base, semS):
        pltpu.async_copy(x_hbm.at[:, pl.ds(base, T)], xv, semS)

    def stage_wait(xv, semS):
        pltpu.make_async_copy(x_hbm.at[:, pl.ds(i32(0), T)], xv, semS).wait()

    def hashc(xv, hv, idxv):
        @plsc.parallel_loop(jnp.int32(0), jnp.int32(GB), jnp.int32(1), unroll=4)
        def hash_step(i):
            for k in range(128 // L):
                off = i * i32(128) + i32(k * L)
                x0 = xv[i32(0), pl.ds(off, L)]
                x1 = xv[i32(1), pl.ds(off, L)]
                x2 = xv[i32(2), pl.ds(off, L)]
                i0 = (x0 * res).astype(i32)
                i1 = (x1 * res).astype(i32)
                i2 = (x2 * res).astype(i32)
                h = (i0 ^ (i1 * P1) ^ (i2 * P2)) & mask
                hv[i, pl.ds(k * L, L)] = h
                q0 = (lax.shift_right_logical(h, i32(7)) * i32(64)
                      + (lax.shift_right_logical(h, i32(3)) & m15))
                for c in range(N_FEATURES):
                    idxv[i32(c * GB) + i, pl.ds(k * L, L)] = q0 + i32(c * 16)

    def fire(idxv, rv, sem):
        def fire_step(j, _):
            for c in range(N_FEATURES):
                pltpu.async_copy(
                    table_hbm.at[idxv.at[i32(c * GB) + j]],
                    rv.at[pl.ds(i32(c * T) + j * i32(128), 128), :],
                    sem,
                )
            return _

        lax.fori_loop(i32(0), i32(GB), fire_step, i32(0))

    def drain(rv, sem):
        pltpu.make_async_copy(
            table_hbm.at[pl.ds(i32(0), N_FEATURES * T), :], rv, sem
        ).wait()

    def extract(hv, rv, outv):
        @plsc.parallel_loop(jnp.int32(0), jnp.int32(T // L), jnp.int32(1), unroll=4)
        def ex_step(i):
            jloc = i * i32(L) + lanes
            h16 = hv[lax.shift_right_logical(i, i32(3)),
                     pl.ds(lax.rem(i, i32(8)) * i32(L), L)]
            sub = h16 & m7
            obase = (lax.div(i, i32(8)) * i32(512)
                     + lax.rem(i, i32(8)) * i32(L))
            for c in range(N_FEATURES):
                val = plsc.load_gather(rv, [i32(c * T) + jloc, sub])
                outv[pl.ds(obase + i32(c * 128), L)] = val

    def outdma(outv, base, sem):
        pltpu.async_copy(outv, out_hbm.at[pl.ds(base * i32(N_FEATURES),
                                                T * N_FEATURES)], sem)

    def outdma_wait(sem):
        pltpu.make_async_copy(
            outvA, out_hbm.at[pl.ds(i32(0), T * N_FEATURES)], sem
        ).wait()

    def pair_step(tt, _):
        base_e = wid * i32(PW) + tt * i32(2 * T)
        base_o = base_e + i32(T)

        stage(xvA, base_e, semSA)
        stage(xvB, base_o, semSB)
        stage_wait(xvA, semSA)
        hashc(xvA, hvA, idxA)

        @pl.when(tt > i32(1))
        def _wait_prev_outB():
            outdma_wait(semOB)

        @pl.when(tt > i32(0))
        def _finish_prev_odd():
            drain(rvB, semB)
            extract(hvB, rvB, outvB)
            outdma(outvB, base_e - i32(T), semOB)

        fire(idxA, rvA, semA)

        stage_wait(xvB, semSB)
        hashc(xvB, hvB, idxB)

        drain(rvA, semA)

        @pl.when(tt > i32(0))
        def _wait_prev_outA():
            outdma_wait(semOA)

        extract(hvA, rvA, outvA)
        outdma(outvA, base_e, semOA)

        fire(idxB, rvB, semB)
        return _

    lax.fori_loop(jnp.int32(0), jnp.int32(N_CHUNKS // 2), pair_step,
                  jnp.int32(0))
    last_base = wid * i32(PW) + i32((N_CHUNKS - 1) * T)
    drain(rvB, semB)

    @pl.when(jnp.int32(N_CHUNKS // 2) > i32(1))
    def _wait_last_outB():
        outdma_wait(semOB)

    outdma_wait(semOA)
    extract(hvB, rvB, outvB)
    outdma(outvB, last_base, semOB)
    outdma_wait(semOB)


@jax.jit
def _run(x, table):
    kfn = functools.partial(
        pl.kernel,
        mesh=plsc.VectorSubcoreMesh(core_axis_name="c", subcore_axis_name="s"),
        compiler_params=pltpu.CompilerParams(
            use_tc_tiling_on_sc=False, needs_layout_passes=False),
        out_type=jax.ShapeDtypeStruct((N_POINTS * N_FEATURES,), jnp.float32),
        scratch_types=[
            pltpu.VMEM((DIM, T), jnp.float32),
            pltpu.VMEM((DIM, T), jnp.float32),
            pltpu.VMEM((GB, 128), jnp.int32),
            pltpu.VMEM((GB, 128), jnp.int32),
            pltpu.VMEM((N_FEATURES * GB, 128), jnp.int32),
            pltpu.VMEM((N_FEATURES * GB, 128), jnp.int32),
            pltpu.VMEM((N_FEATURES * T, 8), jnp.float32),
            pltpu.VMEM((N_FEATURES * T, 8), jnp.float32),
            pltpu.VMEM((T * N_FEATURES,), jnp.float32),
            pltpu.VMEM((T * N_FEATURES,), jnp.float32),
            pltpu.SemaphoreType.DMA,
            pltpu.SemaphoreType.DMA,
            pltpu.SemaphoreType.DMA,
            pltpu.SemaphoreType.DMA,
            pltpu.SemaphoreType.DMA,
            pltpu.SemaphoreType.DMA,
        ],
    )(_body)
    xt = x.T
    chunks = jnp.transpose(
        table.reshape(32768, 128, N_FEATURES), (0, 2, 1)
    ).reshape(N_CHUNK_ROWS, 8)
    out1d = kfn(xt, chunks)
    return jnp.transpose(
        out1d.reshape(N_POINTS // 128, N_FEATURES, 128), (0, 2, 1)
    ).reshape(N_POINTS, N_FEATURES)


def kernel(x, table):
    return _run(x, table)


# speedup vs baseline: 1.2647x; 1.0793x over previous
import functools

import jax
import jax.numpy as jnp
from jax import lax
from jax.experimental import pallas as pl
from jax.experimental.pallas import tpu as pltpu
from jax.experimental.pallas import tpu_sc as plsc

N_POINTS = 1048576
N_FEATURES = 4
DIM = 3
HASH_MASK = 4194304 - 1
RESOLUTION = 512.0
P1 = jnp.int32(2654435761 - (1 << 32))
P2 = jnp.int32(805459861)

NC, NS, L = 2, 16, 16
NW = NC * NS
PW = N_POINTS // NW
T = 1024
N_CHUNKS = PW // T
GB = T // 128
N_CHUNK_ROWS = 4194304 * N_FEATURES // 8


def _body(x_hbm, table_hbm, out_hbm,
          xvA, xvB, hvA, hvB, idxA, idxB, rvA, rvB, outvA, outvB,
          semA, semB, semSA, semSB, semOA, semOB):
    i32 = jnp.int32
    wid = (lax.axis_index("s") * i32(NC) + lax.axis_index("c")).astype(i32)
    mask = jnp.full((L,), HASH_MASK, i32)
    m15 = jnp.full((L,), 15, i32)
    m7 = jnp.full((L,), 7, i32)
    res = jnp.float32(RESOLUTION)
    lanes = lax.iota(i32, L)

    def stage(xv, base, semS):
        pltpu.async_copy(x_hbm.at[:, pl.ds(base, T)], xv, semS)

    def stage_wait(xv, semS):
        pltpu.make_async_copy(x_hbm.at[:, pl.ds(i32(0), T)], xv, semS).wait()

    def hashc(xv, hv, idxv):
        @plsc.parallel_loop(jnp.int32(0), jnp.int32(GB), jnp.int32(1), unroll=4)
        def hash_step(i):
            for k in range(128 // L):
                off = i * i32(128) + i32(k * L)
                x0 = xv[i32(0), pl.ds(off, L)]
                x1 = xv[i32(1), pl.ds(off, L)]
                x2 = xv[i32(2), pl.ds(off, L)]
                i0 = (x0 * res).astype(i32)
                i1 = (x1 * res).astype(i32)
                i2 = (x2 * res).astype(i32)
                h = (i0 ^ (i1 * P1) ^ (i2 * P2)) & mask
                hv[i, pl.ds(k * L, L)] = h
                q0 = (lax.shift_right_logical(h, i32(7)) * i32(64)
                      + (lax.shift_right_logical(h, i32(3)) & m15))
                for c in range(N_FEATURES):
                    idxv[i32(c * GB) + i, pl.ds(k * L, L)] = q0 + i32(c * 16)

    def fire(idxv, rv, sem):
        def fire_step(j, _):
            for c in range(N_FEATURES):
                pltpu.async_copy(
                    table_hbm.at[idxv.at[i32(c * GB) + j]],
                    rv.at[pl.ds(i32(c * T) + j * i32(128), 128), :],
                    sem,
                )
            return _

        lax.fori_loop(i32(0), i32(GB), fire_step, i32(0))

    def drain(rv, sem):
        pltpu.make_async_copy(
            table_hbm.at[pl.ds(i32(0), N_FEATURES * T), :], rv, sem
        ).wait()

    def extract(hv, rv, outv):
        @plsc.parallel_loop(jnp.int32(0), jnp.int32(T // L), jnp.int32(1), unroll=4)
        def ex_step(i):
            jloc = i * i32(L) + lanes
            h16 = hv[lax.shift_right_logical(i, i32(3)),
                     pl.ds(lax.rem(i, i32(8)) * i32(L), L)]
            sub = h16 & m7
            obase = (lax.div(i, i32(8)) * i32(512)
                     + lax.rem(i, i32(8)) * i32(L))
            for c in range(N_FEATURES):
                val = plsc.load_gather(rv, [i32(c * T) + jloc, sub])
                outv[pl.ds(obase + i32(c * 128), L)] = val

    def outdma(outv, base, sem):
        pltpu.async_copy(outv, out_hbm.at[pl.ds(base * i32(N_FEATURES),
                                                T * N_FEATURES)], sem)

    def outdma_wait(sem):
        pltpu.make_async_copy(
            outvA, out_hbm.at[pl.ds(i32(0), T * N_FEATURES)], sem
        ).wait()

    def pair_step(tt, _):
        base_e = wid * i32(PW) + tt * i32(2 * T)
        base_o = base_e + i32(T)

        stage(xvA, base_e, semSA)
        stage(xvB, base_o, semSB)
        stage_wait(xvA, semSA)
        hashc(xvA, hvA, idxA)

        @pl.when(tt > i32(1))
        def _wait_prev_outB():
            outdma_wait(semOB)

        @pl.when(tt > i32(0))
        def _finish_prev_odd():
            drain(rvB, semB)
            extract(hvB, rvB, outvB)
            outdma(outvB, base_e - i32(T), semOB)

        fire(idxA, rvA, semA)

        stage_wait(xvB, semSB)
        hashc(xvB, hvB, idxB)

        fire(idxB, rvB, semB)

        drain(rvA, semA)

        @pl.when(tt > i32(0))
        def _wait_prev_outA():
            outdma_wait(semOA)

        extract(hvA, rvA, outvA)
        outdma(outvA, base_e, semOA)
        return _

    lax.fori_loop(jnp.int32(0), jnp.int32(N_CHUNKS // 2), pair_step,
                  jnp.int32(0))
    last_base = wid * i32(PW) + i32((N_CHUNKS - 1) * T)
    drain(rvB, semB)

    @pl.when(jnp.int32(N_CHUNKS // 2) > i32(1))
    def _wait_last_outB():
        outdma_wait(semOB)

    outdma_wait(semOA)
    extract(hvB, rvB, outvB)
    outdma(outvB, last_base, semOB)
    outdma_wait(semOB)


@jax.jit
def _run(x, table):
    kfn = functools.partial(
        pl.kernel,
        mesh=plsc.VectorSubcoreMesh(core_axis_name="c", subcore_axis_name="s"),
        compiler_params=pltpu.CompilerParams(
            use_tc_tiling_on_sc=False, needs_layout_passes=False),
        out_type=jax.ShapeDtypeStruct((N_POINTS * N_FEATURES,), jnp.float32),
        scratch_types=[
            pltpu.VMEM((DIM, T), jnp.float32),
            pltpu.VMEM((DIM, T), jnp.float32),
            pltpu.VMEM((GB, 128), jnp.int32),
            pltpu.VMEM((GB, 128), jnp.int32),
            pltpu.VMEM((N_FEATURES * GB, 128), jnp.int32),
            pltpu.VMEM((N_FEATURES * GB, 128), jnp.int32),
            pltpu.VMEM((N_FEATURES * T, 8), jnp.float32),
            pltpu.VMEM((N_FEATURES * T, 8), jnp.float32),
            pltpu.VMEM((T * N_FEATURES,), jnp.float32),
            pltpu.VMEM((T * N_FEATURES,), jnp.float32),
            pltpu.SemaphoreType.DMA,
            pltpu.SemaphoreType.DMA,
            pltpu.SemaphoreType.DMA,
            pltpu.SemaphoreType.DMA,
            pltpu.SemaphoreType.DMA,
            pltpu.SemaphoreType.DMA,
        ],
    )(_body)
    xt = x.T
    chunks = jnp.transpose(
        table.reshape(32768, 128, N_FEATURES), (0, 2, 1)
    ).reshape(N_CHUNK_ROWS, 8)
    out1d = kfn(xt, chunks)
    return jnp.transpose(
        out1d.reshape(N_POINTS // 128, N_FEATURES, 128), (0, 2, 1)
    ).reshape(N_POINTS, N_FEATURES)


def kernel(x, table):
    return _run(x, table)
